# baseline (speedup 1.0000x reference)
import dataclasses
import functools

import jax
import jax.numpy as jnp
import numpy as np
from jax import lax
from jax.experimental import pallas as pl
from jax.experimental.pallas import tpu as pltpu
from jax.experimental.pallas import tpu_sc as plsc

N = 10000
E = 320000
DIM = 128

NUM_CORES = 2
NUM_SUBCORES = 16
NUM_WORKERS = NUM_CORES * NUM_SUBCORES
WIN = 96
NUM_WINDOWS = 107
EDGES_PER_WORKER = WIN * NUM_WINDOWS
E_PAD = EDGES_PER_WORKER * NUM_WORKERS
ROWS_PER_SUBCORE = 624


def _filter_body(e_ref, rs_ref, sig_ref, o_ref):
    e = e_ref[...]
    rs = rs_ref[0, 0]
    sig = sig_ref[0, 0]
    g = jnp.exp(-jnp.square(e - rs) / jnp.square(sig))
    w = 0.5 * jnp.cos(np.pi * e)
    o_ref[...] = g * w * (e < 1.0).astype(jnp.float32)


def _edge_filter(e2d, rs, sig):
    return pl.pallas_call(
        _filter_body,
        out_shape=jax.ShapeDtypeStruct(e2d.shape, jnp.float32),
        in_specs=[
            pl.BlockSpec(e2d.shape, lambda: (0, 0)),
            pl.BlockSpec(memory_space=pltpu.SMEM),
            pl.BlockSpec(memory_space=pltpu.SMEM),
        ],
        out_specs=pl.BlockSpec(e2d.shape, lambda: (0, 0)),
    )(e2d, rs, sig)


def _sc_body(h_hbm, pk_hbm, dst_hbm, out_hbm,
             pk0, dst0, rows0, pk1, dst1, rows1,
             pk2, dst2, rows2, acc,
             semg0, semg1, semg2, sems0, sems1, sems2):
    c = lax.axis_index("c")
    s = lax.axis_index("s")
    wid = c * NUM_SUBCORES + s

    @pl.loop(0, WIN)
    def _(i):
        for j in range(DIM // 16):
            rows0[i, pl.ds(j * 16, 16)] = jnp.zeros((16,), jnp.float32)

    rbase = s * ROWS_PER_SUBCORE
    for k in range(ROWS_PER_SUBCORE // WIN):
        pltpu.sync_copy(
            rows0,
            acc.at[pl.ds(rbase + k * WIN, WIN), :],
        )
    rem = ROWS_PER_SUBCORE % WIN
    if rem:
        pltpu.sync_copy(
            rows0.at[pl.ds(0, rem), :],
            acc.at[pl.ds(rbase + ROWS_PER_SUBCORE - rem, rem), :],
        )

    @pl.when(s == NUM_SUBCORES - 1)
    def _():
        pltpu.sync_copy(
            rows0.at[pl.ds(0, 16), :],
            acc.at[pl.ds(NUM_SUBCORES * ROWS_PER_SUBCORE, 16), :],
        )

    plsc.subcore_barrier()

    base = wid * EDGES_PER_WORKER

    def stage(w, pk_v, dst_v, rows_v, semg):
        off2 = (base + w * WIN) * 2
        pltpu.sync_copy(pk_hbm.at[pl.ds(off2, 2 * WIN)], pk_v)
        pltpu.sync_copy(dst_hbm.at[pl.ds(base + w * WIN, WIN)], dst_v)
        pltpu.async_copy(h_hbm.at[pk_v.at[pl.ds(0, WIN)]], rows_v, semg)

    def refill(w, pk_v, dst_v, rows_v, semg, sems):
        pltpu.make_async_copy(rows_v, acc.at[dst_v], sems).wait()
        stage(w, pk_v, dst_v, rows_v, semg)

    def process(pk_v, dst_v, rows_v, semg, sems):
        pltpu.make_async_copy(
            h_hbm.at[pk_v.at[pl.ds(0, WIN)]], rows_v, semg
        ).wait()

        @pl.loop(0, WIN // 16)
        def _(g):
            fvec = plsc.bitcast(pk_v[pl.ds(WIN + g * 16, 16)], jnp.float32)
            for l in range(16):
                fv = fvec[l]
                row = g * 16 + l
                for j in range(DIM // 16):
                    sl = pl.ds(j * 16, 16)
                    rows_v[row, sl] = rows_v[row, sl] * fv

        pltpu.async_copy(rows_v, acc.at[dst_v], sems, add=True)

    A = (pk0, dst0, rows0, semg0, sems0)
    B = (pk1, dst1, rows1, semg1, sems1)
    C = (pk2, dst2, rows2, semg2, sems2)

    stage(0, *A[:4])
    stage(1, *B[:4])
    stage(2, *C[:4])

    @pl.loop(0, (NUM_WINDOWS - 2) // 3)
    def _(p):
        w0 = 3 * p
        process(*A)
        process(*B)
        refill(w0 + 3, *A)
        process(*C)
        refill(w0 + 4, *B)

        @pl.when(w0 + 5 < NUM_WINDOWS)
        def _():
            refill(w0 + 5, *C)

    process(*A)
    process(*B)

    pltpu.make_async_copy(rows2, acc.at[dst2], sems2).wait()
    pltpu.make_async_copy(rows0, acc.at[dst0], sems0).wait()
    pltpu.make_async_copy(rows1, acc.at[dst1], sems1).wait()

    plsc.subcore_barrier()

    pltpu.sync_copy(
        acc.at[pl.ds(rbase, ROWS_PER_SUBCORE), :],
        out_hbm.at[c, pl.ds(rbase, ROWS_PER_SUBCORE), :],
    )

    @pl.when(s == NUM_SUBCORES - 1)
    def _():
        pltpu.sync_copy(
            acc.at[pl.ds(NUM_SUBCORES * ROWS_PER_SUBCORE, 16), :],
            out_hbm.at[c, pl.ds(NUM_SUBCORES * ROWS_PER_SUBCORE, 16), :],
        )


def _sc_scatter(h, packed, dst):
    mesh = plsc.VectorSubcoreMesh(core_axis_name="c", subcore_axis_name="s")
    cp = pltpu.CompilerParams()
    if "needs_layout_passes" in pltpu.CompilerParams.__dataclass_fields__:
        cp = dataclasses.replace(cp, needs_layout_passes=False)
    kern = pl.kernel(
        _sc_body,
        out_type=jax.ShapeDtypeStruct((NUM_CORES, N, DIM), jnp.float32),
        compiler_params=cp,
        mesh=mesh,
        scratch_types=[
            pltpu.VMEM((2 * WIN,), jnp.int32),
            pltpu.VMEM((WIN,), jnp.int32),
            pltpu.VMEM((WIN, DIM), jnp.float32),
            pltpu.VMEM((2 * WIN,), jnp.int32),
            pltpu.VMEM((WIN,), jnp.int32),
            pltpu.VMEM((WIN, DIM), jnp.float32),
            pltpu.VMEM((2 * WIN,), jnp.int32),
            pltpu.VMEM((WIN,), jnp.int32),
            pltpu.VMEM((WIN, DIM), jnp.float32),
            pltpu.VMEM_SHARED((N, DIM), jnp.float32),
            pltpu.SemaphoreType.DMA,
            pltpu.SemaphoreType.DMA,
            pltpu.SemaphoreType.DMA,
            pltpu.SemaphoreType.DMA,
            pltpu.SemaphoreType.DMA,
            pltpu.SemaphoreType.DMA,
        ],
    )
    return kern(h, packed, dst)


ROW_BLK = 1000


def _linear_body(p_ref, v_ref, wt_ref, b_ref, o_ref):
    x = p_ref[0] + p_ref[1] + v_ref[...]
    y = jnp.dot(x, wt_ref[...], preferred_element_type=jnp.float32)
    o_ref[...] = jnp.maximum(y + b_ref[...], 0.0)


def _linear_relu(p, v, wt, b2d):
    return pl.pallas_call(
        _linear_body,
        grid=(N // ROW_BLK,),
        out_shape=jax.ShapeDtypeStruct((N, DIM), jnp.float32),
        in_specs=[
            pl.BlockSpec((NUM_CORES, ROW_BLK, DIM), lambda i: (0, i, 0)),
            pl.BlockSpec((ROW_BLK, DIM), lambda i: (i, 0)),
            pl.BlockSpec((DIM, DIM), lambda i: (0, 0)),
            pl.BlockSpec((1, DIM), lambda i: (0, 0)),
        ],
        out_specs=pl.BlockSpec((ROW_BLK, DIM), lambda i: (i, 0)),
    )(p, v, wt, b2d)


def kernel(v, e, rs, sigma, W, b, edge_index):
    src = edge_index[0]
    dst = edge_index[1]
    f2d = _edge_filter(
        e.reshape(E // DIM, DIM),
        rs.reshape(1, 1),
        sigma.reshape(1, 1),
    )
    pad = E_PAD - E
    f = jnp.pad(f2d.reshape(E), (0, pad))
    src = jnp.pad(src, (0, pad))
    dst = jnp.pad(dst, (0, pad))
    fbits = lax.bitcast_convert_type(f, jnp.int32).reshape(E_PAD // WIN, WIN)
    packed = jnp.concatenate(
        [src.reshape(E_PAD // WIN, WIN), fbits], axis=1
    ).reshape(-1)

    wt = W.T
    b2d = b.reshape(1, DIM)

    h = v
    for _ in range(3):
        p = _sc_scatter(h, packed, dst)
        h = _linear_relu(p, v, wt, b2d)
    return h


# speedup vs baseline: 2.3519x; 2.3519x over previous
import dataclasses
import functools

import jax
import jax.numpy as jnp
import numpy as np
from jax import lax
from jax.experimental import pallas as pl
from jax.experimental.pallas import tpu as pltpu
from jax.experimental.pallas import tpu_sc as plsc

N = 10000
E = 320000
DIM = 128

NUM_CORES = 2
NUM_SUBCORES = 16
NUM_WORKERS = NUM_CORES * NUM_SUBCORES
EDGES_PER_WORKER = E // NUM_WORKERS
WIN = 80
NUM_WINDOWS = EDGES_PER_WORKER // WIN
ROWS_PER_SUBCORE = 624


def _filter_body(e_ref, rs_ref, sig_ref, o_ref):
    e = e_ref[...]
    rs = rs_ref[0, 0]
    sig = sig_ref[0, 0]
    g = jnp.exp(-jnp.square(e - rs) / jnp.square(sig))
    w = 0.5 * jnp.cos(np.pi * e)
    o_ref[...] = g * w * (e < 1.0).astype(jnp.float32)


def _edge_filter(e2d, rs, sig):
    return pl.pallas_call(
        _filter_body,
        out_shape=jax.ShapeDtypeStruct(e2d.shape, jnp.float32),
        in_specs=[
            pl.BlockSpec(e2d.shape, lambda: (0, 0)),
            pl.BlockSpec(memory_space=pltpu.SMEM),
            pl.BlockSpec(memory_space=pltpu.SMEM),
        ],
        out_specs=pl.BlockSpec(e2d.shape, lambda: (0, 0)),
    )(e2d, rs, sig)


def _sc_body(h_hbm, pk_hbm, dst_hbm, out_hbm,
             pk0, dst0, rows0, pk1, dst1, rows1,
             pk2, dst2, rows2, acc,
             semg0, semg1, semg2, sems0, sems1, sems2):
    c = lax.axis_index("c")
    s = lax.axis_index("s")
    wid = c * NUM_SUBCORES + s

    @pl.loop(0, WIN)
    def _(i):
        for j in range(DIM // 16):
            rows0[i, pl.ds(j * 16, 16)] = jnp.zeros((16,), jnp.float32)

    rbase = s * ROWS_PER_SUBCORE
    for k in range(ROWS_PER_SUBCORE // WIN):
        pltpu.sync_copy(
            rows0,
            acc.at[pl.ds(rbase + k * WIN, WIN), :],
        )
    pltpu.sync_copy(
        rows0.at[pl.ds(0, 64), :],
        acc.at[pl.ds(rbase + 560, 64), :],
    )

    @pl.when(s == NUM_SUBCORES - 1)
    def _():
        pltpu.sync_copy(
            rows0.at[pl.ds(0, 16), :],
            acc.at[pl.ds(NUM_SUBCORES * ROWS_PER_SUBCORE, 16), :],
        )

    plsc.subcore_barrier()

    base = wid * EDGES_PER_WORKER

    def stage(w, pk_v, dst_v, rows_v, semg):
        off2 = (base + w * WIN) * 2
        pltpu.sync_copy(pk_hbm.at[pl.ds(off2, 2 * WIN)], pk_v)
        pltpu.sync_copy(dst_hbm.at[pl.ds(base + w * WIN, WIN)], dst_v)
        pltpu.async_copy(h_hbm.at[pk_v.at[pl.ds(0, WIN)]], rows_v, semg)

    def refill(w, pk_v, dst_v, rows_v, semg, sems):
        pltpu.make_async_copy(rows_v, acc.at[dst_v], sems).wait()
        stage(w, pk_v, dst_v, rows_v, semg)

    def process(pk_v, dst_v, rows_v, semg, sems):
        pltpu.make_async_copy(
            h_hbm.at[pk_v.at[pl.ds(0, WIN)]], rows_v, semg
        ).wait()

        @pl.loop(0, WIN // 16)
        def _(g):
            fvec = plsc.bitcast(pk_v[pl.ds(WIN + g * 16, 16)], jnp.float32)
            for l in range(16):
                fv = fvec[l]
                row = g * 16 + l
                for j in range(DIM // 16):
                    sl = pl.ds(j * 16, 16)
                    rows_v[row, sl] = rows_v[row, sl] * fv

        pltpu.async_copy(rows_v, acc.at[dst_v], sems, add=True)

    A = (pk0, dst0, rows0, semg0, sems0)
    B = (pk1, dst1, rows1, semg1, sems1)
    C = (pk2, dst2, rows2, semg2, sems2)

    stage(0, *A[:4])
    stage(1, *B[:4])
    stage(2, *C[:4])

    @pl.loop(0, (NUM_WINDOWS - 2) // 3)
    def _(p):
        w0 = 3 * p
        process(*A)
        process(*B)
        refill(w0 + 3, *A)
        process(*C)
        refill(w0 + 4, *B)

        @pl.when(w0 + 5 < NUM_WINDOWS)
        def _():
            refill(w0 + 5, *C)

    process(*A)
    process(*B)

    pltpu.make_async_copy(rows2, acc.at[dst2], sems2).wait()
    pltpu.make_async_copy(rows0, acc.at[dst0], sems0).wait()
    pltpu.make_async_copy(rows1, acc.at[dst1], sems1).wait()

    plsc.subcore_barrier()

    pltpu.sync_copy(
        acc.at[pl.ds(rbase, ROWS_PER_SUBCORE), :],
        out_hbm.at[c, pl.ds(rbase, ROWS_PER_SUBCORE), :],
    )

    @pl.when(s == NUM_SUBCORES - 1)
    def _():
        pltpu.sync_copy(
            acc.at[pl.ds(NUM_SUBCORES * ROWS_PER_SUBCORE, 16), :],
            out_hbm.at[c, pl.ds(NUM_SUBCORES * ROWS_PER_SUBCORE, 16), :],
        )


def _sc_scatter(h, packed, dst):
    mesh = plsc.VectorSubcoreMesh(core_axis_name="c", subcore_axis_name="s")
    cp = pltpu.CompilerParams()
    if "needs_layout_passes" in pltpu.CompilerParams.__dataclass_fields__:
        cp = dataclasses.replace(cp, needs_layout_passes=False)
    kern = pl.kernel(
        _sc_body,
        out_type=jax.ShapeDtypeStruct((NUM_CORES, N, DIM), jnp.float32),
        compiler_params=cp,
        mesh=mesh,
        scratch_types=[
            pltpu.VMEM((2 * WIN,), jnp.int32),
            pltpu.VMEM((WIN,), jnp.int32),
            pltpu.VMEM((WIN, DIM), jnp.float32),
            pltpu.VMEM((2 * WIN,), jnp.int32),
            pltpu.VMEM((WIN,), jnp.int32),
            pltpu.VMEM((WIN, DIM), jnp.float32),
            pltpu.VMEM((2 * WIN,), jnp.int32),
            pltpu.VMEM((WIN,), jnp.int32),
            pltpu.VMEM((WIN, DIM), jnp.float32),
            pltpu.VMEM_SHARED((N, DIM), jnp.float32),
            pltpu.SemaphoreType.DMA,
            pltpu.SemaphoreType.DMA,
            pltpu.SemaphoreType.DMA,
            pltpu.SemaphoreType.DMA,
            pltpu.SemaphoreType.DMA,
            pltpu.SemaphoreType.DMA,
        ],
    )
    return kern(h, packed, dst)


ROW_BLK = 1000


def _linear_body(p_ref, v_ref, wt_ref, b_ref, o_ref):
    x = p_ref[0] + p_ref[1] + v_ref[...]
    y = jnp.dot(x, wt_ref[...], preferred_element_type=jnp.float32)
    o_ref[...] = jnp.maximum(y + b_ref[...], 0.0)


def _linear_relu(p, v, wt, b2d):
    return pl.pallas_call(
        _linear_body,
        grid=(N // ROW_BLK,),
        out_shape=jax.ShapeDtypeStruct((N, DIM), jnp.float32),
        in_specs=[
            pl.BlockSpec((NUM_CORES, ROW_BLK, DIM), lambda i: (0, i, 0)),
            pl.BlockSpec((ROW_BLK, DIM), lambda i: (i, 0)),
            pl.BlockSpec((DIM, DIM), lambda i: (0, 0)),
            pl.BlockSpec((1, DIM), lambda i: (0, 0)),
        ],
        out_specs=pl.BlockSpec((ROW_BLK, DIM), lambda i: (i, 0)),
    )(p, v, wt, b2d)


def kernel(v, e, rs, sigma, W, b, edge_index):
    src = edge_index[0]
    dst = edge_index[1]
    f2d = _edge_filter(
        e.reshape(E // DIM, DIM),
        rs.reshape(1, 1),
        sigma.reshape(1, 1),
    )
    f = f2d.reshape(E)
    fbits = lax.bitcast_convert_type(f, jnp.int32).reshape(E // WIN, WIN)
    packed = jnp.concatenate(
        [src.reshape(E // WIN, WIN), fbits], axis=1
    ).reshape(-1)

    wt = W.T
    b2d = b.reshape(1, DIM)

    h = v
    for _ in range(3):
        p = _sc_scatter(h, packed, dst)
        h = _linear_relu(p, v, wt, b2d)
    return h
